# baseline (speedup 1.0000x reference)
import functools

import jax
import jax.numpy as jnp
from jax import lax
from jax.experimental import pallas as pl
from jax.experimental.pallas import tpu as pltpu
from jax.experimental.pallas import tpu_sc as plsc

E = 8
TOPK = 2
D = 1024
FFN = 1024
S = 2048

BK = 256
NA = TOPK * S
NB = NA // BK + E
NP = NB * BK
NW = 32
APW = NA // NW
CH = 32
NCH = APW // CH

_LANES = 128


def _router_kernel(x_ref, rw_ref, w_ref, e0_ref, e1_ref, tw0_ref, tw1_ref):
    x = x_ref[...]
    logits = jnp.dot(x, rw_ref[...], preferred_element_type=jnp.float32)
    lane = lax.broadcasted_iota(jnp.int32, logits.shape, 1)
    real = lane < E
    logits = jnp.where(real, logits, jnp.float32(-1e30))
    m = jnp.max(logits, axis=1, keepdims=True)
    p = jnp.where(real, jnp.exp(logits - m), 0.0)
    s = jnp.sum(p, axis=1, keepdims=True)
    w = p / s
    w_ref[...] = w[:, :E]
    w0 = jnp.max(w, axis=1, keepdims=True)
    e0 = jnp.min(jnp.where((w == w0) & real, lane, 2 * _LANES),
                 axis=1, keepdims=True)
    wm = jnp.where(lane == e0, jnp.float32(-1.0), w)
    w1v = jnp.max(wm, axis=1, keepdims=True)
    e1 = jnp.min(jnp.where((wm == w1v) & real, lane, 2 * _LANES),
                 axis=1, keepdims=True)
    tot = w0 + w1v
    e0_ref[...] = e0
    e1_ref[...] = e1
    tw0_ref[...] = w0 / tot
    tw1_ref[...] = w1v / tot


def _run_router(x2, router_w):
    rw_pad = jnp.zeros((D, _LANES), jnp.float32).at[:, :E].set(router_w)
    bs = 256
    return pl.pallas_call(
        _router_kernel,
        grid=(S // bs,),
        in_specs=[
            pl.BlockSpec((bs, D), lambda i: (i, 0)),
            pl.BlockSpec((D, _LANES), lambda i: (0, 0)),
        ],
        out_specs=[
            pl.BlockSpec((bs, E), lambda i: (i, 0)),
            pl.BlockSpec((bs, 1), lambda i: (i, 0)),
            pl.BlockSpec((bs, 1), lambda i: (i, 0)),
            pl.BlockSpec((bs, 1), lambda i: (i, 0)),
            pl.BlockSpec((bs, 1), lambda i: (i, 0)),
        ],
        out_shape=[
            jax.ShapeDtypeStruct((S, E), jnp.float32),
            jax.ShapeDtypeStruct((S, 1), jnp.int32),
            jax.ShapeDtypeStruct((S, 1), jnp.int32),
            jax.ShapeDtypeStruct((S, 1), jnp.float32),
            jax.ShapeDtypeStruct((S, 1), jnp.float32),
        ],
        compiler_params=pltpu.CompilerParams(
            dimension_semantics=("parallel",)),
    )(x2, rw_pad)


def _routing_metadata(e0, e1):
    e_flat = jnp.concatenate([e0, e1], axis=1).reshape(NA)
    onehot = (e_flat[:, None] == jnp.arange(E)[None, :]).astype(jnp.int32)
    csum = jnp.cumsum(onehot, axis=0)
    rank = jnp.sum(onehot * csum, axis=1) - 1
    counts = csum[-1]
    nb_e = ((counts + BK - 1) // BK).astype(jnp.int32)
    blk_start = (jnp.cumsum(nb_e) - nb_e).astype(jnp.int32)
    start_a = jnp.sum(onehot * blk_start[None, :], axis=1)
    jpos = (start_a * BK + rank).astype(jnp.int32)
    tok = (jnp.arange(NA, dtype=jnp.int32) // TOPK)
    return blk_start, nb_e, jpos, tok


_SC_MESH = dict(core_axis_name="c", subcore_axis_name="s")


def _sc_dispatch(x_f32, tok, jpos3):
    @functools.partial(
        pl.kernel,
        out_type=jax.ShapeDtypeStruct((NP, D), jnp.float32),
        mesh=plsc.VectorSubcoreMesh(**_SC_MESH),
        scratch_types=[
            pltpu.VMEM((APW,), jnp.int32),
            pltpu.VMEM((NCH, CH), jnp.int32),
            pltpu.VMEM((CH, D), jnp.float32),
            pltpu.VMEM((CH, D), jnp.float32),
            pltpu.SemaphoreType.DMA,
            pltpu.SemaphoreType.DMA,
            pltpu.SemaphoreType.DMA,
            pltpu.SemaphoreType.DMA,
        ],
    )
    def k(x_hbm, tok_hbm, jpos_hbm, out_hbm, tok_v, j_v, buf0, buf1,
          sg0, sg1, ss0, ss1):
        wid = lax.axis_index("s") * 2 + lax.axis_index("c")
        base = wid * APW
        pltpu.sync_copy(tok_hbm.at[pl.ds(base, APW)], tok_v)
        pltpu.sync_copy(jpos_hbm.at[wid], j_v)

        bufs = (buf0, buf1)
        gsems = (sg0, sg1)
        ssems = (ss0, ss1)
        gets = [None, None]
        puts = [None, None]
        for c in range(NCH):
            p = c % 2
            if puts[p] is not None:
                puts[p].wait()
            gets[p] = pltpu.async_copy(
                x_hbm.at[tok_v.at[pl.ds(c * CH, CH)]], bufs[p], gsems[p])
            if c > 0:
                q = (c - 1) % 2
                gets[q].wait()
                puts[q] = pltpu.async_copy(
                    bufs[q], out_hbm.at[j_v.at[c - 1]], ssems[q])
        last = (NCH - 1) % 2
        gets[last].wait()
        puts[last] = pltpu.async_copy(
            bufs[last], out_hbm.at[j_v.at[NCH - 1]], ssems[last])
        puts[0].wait()
        puts[1].wait()

    return k(x_f32, tok, jpos3)


def _glu_kernel(bs_ref, ne_ref, xg_ref, w1_ref, v1_ref, w2_ref, y_ref):
    e = pl.program_id(0)
    j = pl.program_id(1)

    @pl.when(j < ne_ref[e])
    def _():
        xb = xg_ref[...]
        h1 = lax.dot_general(xb, w1_ref[0], (((1,), (1,)), ((), ())),
                             preferred_element_type=jnp.float32,
                             precision=lax.Precision.DEFAULT)
        h2 = lax.dot_general(xb, v1_ref[0], (((1,), (1,)), ((), ())),
                             preferred_element_type=jnp.float32,
                             precision=lax.Precision.DEFAULT)
        g = h1 * lax.logistic(h1) * h2
        y = lax.dot_general(g, w2_ref[0], (((1,), (0,)), ((), ())),
                            preferred_element_type=jnp.float32,
                            precision=lax.Precision.DEFAULT)
        y_ref[...] = y


MAXB = S // BK


def _run_glu(blk_start, nb_e, xg, w1r, v1r, w2r):
    def _row_map(e, j, bs, ne):
        return (bs[e] + jnp.maximum(jnp.minimum(j, ne[e] - 1), 0), 0)

    def _w_map(e, j, bs, ne):
        return (e, 0, 0)

    def _out_map(e, j, bs, ne):
        return (jnp.where(j < ne[e], bs[e] + jnp.minimum(j, MAXB - 1), NB), 0)

    grid_spec = pltpu.PrefetchScalarGridSpec(
        num_scalar_prefetch=2,
        grid=(E, MAXB),
        in_specs=[
            pl.BlockSpec((BK, D), _row_map),
            pl.BlockSpec((1, FFN, D), _w_map),
            pl.BlockSpec((1, FFN, D), _w_map),
            pl.BlockSpec((1, FFN, D), _w_map),
        ],
        out_specs=pl.BlockSpec((BK, D), _out_map),
    )
    return pl.pallas_call(
        _glu_kernel,
        grid_spec=grid_spec,
        out_shape=jax.ShapeDtypeStruct((NP + BK, D), jnp.float32),
    )(blk_start, nb_e, xg, w1r, v1r, w2r)


def _sc_collect(y, jcat):
    rpw = TOPK * S // NW
    nch = rpw // CH

    @functools.partial(
        pl.kernel,
        out_type=jax.ShapeDtypeStruct((TOPK * S, D), jnp.float32),
        mesh=plsc.VectorSubcoreMesh(**_SC_MESH),
        scratch_types=[
            pltpu.VMEM((rpw,), jnp.int32),
            pltpu.VMEM((CH, D), jnp.float32),
            pltpu.VMEM((CH, D), jnp.float32),
            pltpu.SemaphoreType.DMA,
            pltpu.SemaphoreType.DMA,
            pltpu.SemaphoreType.DMA,
            pltpu.SemaphoreType.DMA,
        ],
    )
    def k(y_hbm, j_hbm, out_hbm, j_v, buf0, buf1, sg0, sg1, ss0, ss1):
        wid = lax.axis_index("s") * 2 + lax.axis_index("c")
        base = wid * rpw
        pltpu.sync_copy(j_hbm.at[pl.ds(base, rpw)], j_v)

        bufs = (buf0, buf1)
        gsems = (sg0, sg1)
        ssems = (ss0, ss1)
        gets = [None, None]
        puts = [None, None]
        for c in range(nch):
            p = c % 2
            if puts[p] is not None:
                puts[p].wait()
            gets[p] = pltpu.async_copy(
                y_hbm.at[j_v.at[pl.ds(c * CH, CH)]], bufs[p], gsems[p])
            if c > 0:
                q = (c - 1) % 2
                gets[q].wait()
                puts[q] = pltpu.async_copy(
                    bufs[q], out_hbm.at[pl.ds(base + (c - 1) * CH, CH)],
                    ssems[q])
        last = (nch - 1) % 2
        gets[last].wait()
        puts[last] = pltpu.async_copy(
            bufs[last], out_hbm.at[pl.ds(base + (nch - 1) * CH, CH)],
            ssems[last])
        puts[0].wait()
        puts[1].wait()

    return k(y, jcat)


def _combine_kernel(a_ref, b_ref, tw0_ref, tw1_ref, o_ref):
    o_ref[...] = tw0_ref[...] * a_ref[...] + tw1_ref[...] * b_ref[...]


def _run_combine(yg, tw0, tw1):
    bs = 256
    return pl.pallas_call(
        _combine_kernel,
        grid=(S // bs,),
        in_specs=[
            pl.BlockSpec((bs, D), lambda i: (i, 0)),
            pl.BlockSpec((bs, D), lambda i: (i + S // bs, 0)),
            pl.BlockSpec((bs, 1), lambda i: (i, 0)),
            pl.BlockSpec((bs, 1), lambda i: (i, 0)),
        ],
        out_specs=pl.BlockSpec((bs, D), lambda i: (i, 0)),
        out_shape=jax.ShapeDtypeStruct((S, D), jnp.float32),
        compiler_params=pltpu.CompilerParams(
            dimension_semantics=("parallel",)),
    )(yg, yg, tw0, tw1)


def kernel(x, router_w, w1, v1, w2):
    x2 = x.reshape(S, D)
    weights, e0, e1, tw0, tw1 = _run_router(x2, router_w)
    blk_start, nb_e, jpos, tok = _routing_metadata(e0, e1)

    xg = _sc_dispatch(x2, tok, jpos.reshape(NW, NCH, CH))

    w1r = w1.reshape(E, FFN, D)
    v1r = v1.reshape(E, FFN, D)
    w2r = w2.reshape(E, FFN, D)
    y = _run_glu(blk_start, nb_e, xg, w1r, v1r, w2r)

    jcat = jpos.reshape(S, TOPK).T.reshape(TOPK * S)
    yg = _sc_collect(y, jcat)

    out = _run_combine(yg, tw0, tw1)
    return out.reshape(1, S, D), weights.reshape(1, S, E)


# speedup vs baseline: 1.1833x; 1.1833x over previous
import functools

import jax
import jax.numpy as jnp
from jax import lax
from jax.experimental import pallas as pl
from jax.experimental.pallas import tpu as pltpu
from jax.experimental.pallas import tpu_sc as plsc

E = 8
TOPK = 2
D = 1024
FFN = 1024
S = 2048

BK = 512
NA = TOPK * S
NB = NA // BK + E
NP = NB * BK
NW = 32
APW = NA // NW
CH = 32
NCH = APW // CH

_LANES = 128


def _router_kernel(x_ref, rw_ref, w_ref, e0_ref, e1_ref, tw0_ref, tw1_ref):
    x = x_ref[...]
    logits = jnp.dot(x, rw_ref[...], preferred_element_type=jnp.float32)
    lane = lax.broadcasted_iota(jnp.int32, logits.shape, 1)
    real = lane < E
    logits = jnp.where(real, logits, jnp.float32(-1e30))
    m = jnp.max(logits, axis=1, keepdims=True)
    p = jnp.where(real, jnp.exp(logits - m), 0.0)
    s = jnp.sum(p, axis=1, keepdims=True)
    w = p / s
    w_ref[...] = w[:, :E]
    w0 = jnp.max(w, axis=1, keepdims=True)
    e0 = jnp.min(jnp.where((w == w0) & real, lane, 2 * _LANES),
                 axis=1, keepdims=True)
    wm = jnp.where(lane == e0, jnp.float32(-1.0), w)
    w1v = jnp.max(wm, axis=1, keepdims=True)
    e1 = jnp.min(jnp.where((wm == w1v) & real, lane, 2 * _LANES),
                 axis=1, keepdims=True)
    tot = w0 + w1v
    e0_ref[...] = e0
    e1_ref[...] = e1
    tw0_ref[...] = w0 / tot
    tw1_ref[...] = w1v / tot


def _run_router(x2, router_w):
    rw_pad = jnp.zeros((D, _LANES), jnp.float32).at[:, :E].set(router_w)
    bs = 256
    return pl.pallas_call(
        _router_kernel,
        grid=(S // bs,),
        in_specs=[
            pl.BlockSpec((bs, D), lambda i: (i, 0)),
            pl.BlockSpec((D, _LANES), lambda i: (0, 0)),
        ],
        out_specs=[
            pl.BlockSpec((bs, E), lambda i: (i, 0)),
            pl.BlockSpec((bs, 1), lambda i: (i, 0)),
            pl.BlockSpec((bs, 1), lambda i: (i, 0)),
            pl.BlockSpec((bs, 1), lambda i: (i, 0)),
            pl.BlockSpec((bs, 1), lambda i: (i, 0)),
        ],
        out_shape=[
            jax.ShapeDtypeStruct((S, E), jnp.float32),
            jax.ShapeDtypeStruct((S, 1), jnp.int32),
            jax.ShapeDtypeStruct((S, 1), jnp.int32),
            jax.ShapeDtypeStruct((S, 1), jnp.float32),
            jax.ShapeDtypeStruct((S, 1), jnp.float32),
        ],
        compiler_params=pltpu.CompilerParams(
            dimension_semantics=("parallel",)),
    )(x2, rw_pad)


def _routing_metadata(e0, e1):
    e_flat = jnp.concatenate([e0, e1], axis=1).reshape(NA)
    onehot = (e_flat[:, None] == jnp.arange(E)[None, :]).astype(jnp.int32)
    csum = jnp.cumsum(onehot, axis=0)
    rank = jnp.sum(onehot * csum, axis=1) - 1
    counts = csum[-1]
    nb_e = ((counts + BK - 1) // BK).astype(jnp.int32)
    blk_start = (jnp.cumsum(nb_e) - nb_e).astype(jnp.int32)
    nb_used = jnp.sum(nb_e).astype(jnp.int32).reshape(1)
    block_expert = (jnp.sum(
        (jnp.arange(NB)[:, None] >= blk_start[None, :]).astype(jnp.int32),
        axis=1) - 1).astype(jnp.int32)
    start_a = jnp.sum(onehot * blk_start[None, :], axis=1)
    jpos = (start_a * BK + rank).astype(jnp.int32)
    tok = (jnp.arange(NA, dtype=jnp.int32) // TOPK)
    return block_expert, nb_used, jpos, tok


_SC_MESH = dict(core_axis_name="c", subcore_axis_name="s")


def _sc_dispatch(x_f32, tok, jpos3):
    @functools.partial(
        pl.kernel,
        out_type=jax.ShapeDtypeStruct((NP, D), jnp.float32),
        mesh=plsc.VectorSubcoreMesh(**_SC_MESH),
        scratch_types=[
            pltpu.VMEM((APW,), jnp.int32),
            pltpu.VMEM((NCH, CH), jnp.int32),
            pltpu.VMEM((CH, D), jnp.float32),
            pltpu.VMEM((CH, D), jnp.float32),
            pltpu.SemaphoreType.DMA,
            pltpu.SemaphoreType.DMA,
            pltpu.SemaphoreType.DMA,
            pltpu.SemaphoreType.DMA,
        ],
    )
    def k(x_hbm, tok_hbm, jpos_hbm, out_hbm, tok_v, j_v, buf0, buf1,
          sg0, sg1, ss0, ss1):
        wid = lax.axis_index("s") * 2 + lax.axis_index("c")
        base = wid * APW
        pltpu.sync_copy(tok_hbm.at[pl.ds(base, APW)], tok_v)
        pltpu.sync_copy(jpos_hbm.at[wid], j_v)

        bufs = (buf0, buf1)
        gsems = (sg0, sg1)
        ssems = (ss0, ss1)
        gets = [None, None]
        puts = [None, None]
        for c in range(NCH):
            p = c % 2
            if puts[p] is not None:
                puts[p].wait()
            gets[p] = pltpu.async_copy(
                x_hbm.at[tok_v.at[pl.ds(c * CH, CH)]], bufs[p], gsems[p])
            if c > 0:
                q = (c - 1) % 2
                gets[q].wait()
                puts[q] = pltpu.async_copy(
                    bufs[q], out_hbm.at[j_v.at[c - 1]], ssems[q])
        last = (NCH - 1) % 2
        gets[last].wait()
        puts[last] = pltpu.async_copy(
            bufs[last], out_hbm.at[j_v.at[NCH - 1]], ssems[last])
        puts[0].wait()
        puts[1].wait()

    return k(x_f32, tok, jpos3)


def _glu_kernel(be_ref, nu_ref, xg_ref, w1_ref, v1_ref, w2_ref, y_ref):
    b = pl.program_id(0)

    @pl.when(b < nu_ref[0])
    def _():
        xb = xg_ref[...]
        h1 = lax.dot_general(xb, w1_ref[0], (((1,), (1,)), ((), ())),
                             preferred_element_type=jnp.float32,
                             precision=lax.Precision.DEFAULT)
        h2 = lax.dot_general(xb, v1_ref[0], (((1,), (1,)), ((), ())),
                             preferred_element_type=jnp.float32,
                             precision=lax.Precision.DEFAULT)
        g = h1 * lax.logistic(h1) * h2
        y = lax.dot_general(g, w2_ref[0], (((1,), (0,)), ((), ())),
                            preferred_element_type=jnp.float32,
                            precision=lax.Precision.DEFAULT)
        y_ref[...] = y


def _run_glu(block_expert, nb_used, xg, w1r, v1r, w2r):
    def _row_map(b, be, nu):
        return (jnp.minimum(b, nu[0] - 1), 0)

    def _w_map(b, be, nu):
        return (be[jnp.minimum(b, nu[0] - 1)], 0, 0)

    def _out_map(b, be, nu):
        return (jnp.where(b < nu[0], b, NB), 0)

    grid_spec = pltpu.PrefetchScalarGridSpec(
        num_scalar_prefetch=2,
        grid=(NB,),
        in_specs=[
            pl.BlockSpec((BK, D), _row_map),
            pl.BlockSpec((1, FFN, D), _w_map),
            pl.BlockSpec((1, FFN, D), _w_map),
            pl.BlockSpec((1, FFN, D), _w_map),
        ],
        out_specs=pl.BlockSpec((BK, D), _out_map),
    )
    return pl.pallas_call(
        _glu_kernel,
        grid_spec=grid_spec,
        out_shape=jax.ShapeDtypeStruct((NP + BK, D), jnp.float32),
    )(block_expert, nb_used, xg, w1r, v1r, w2r)


def _sc_collect(y, jcat):
    rpw = TOPK * S // NW
    nch = rpw // CH

    @functools.partial(
        pl.kernel,
        out_type=jax.ShapeDtypeStruct((TOPK * S, D), jnp.float32),
        mesh=plsc.VectorSubcoreMesh(**_SC_MESH),
        scratch_types=[
            pltpu.VMEM((rpw,), jnp.int32),
            pltpu.VMEM((CH, D), jnp.float32),
            pltpu.VMEM((CH, D), jnp.float32),
            pltpu.SemaphoreType.DMA,
            pltpu.SemaphoreType.DMA,
            pltpu.SemaphoreType.DMA,
            pltpu.SemaphoreType.DMA,
        ],
    )
    def k(y_hbm, j_hbm, out_hbm, j_v, buf0, buf1, sg0, sg1, ss0, ss1):
        wid = lax.axis_index("s") * 2 + lax.axis_index("c")
        base = wid * rpw
        pltpu.sync_copy(j_hbm.at[pl.ds(base, rpw)], j_v)

        bufs = (buf0, buf1)
        gsems = (sg0, sg1)
        ssems = (ss0, ss1)
        gets = [None, None]
        puts = [None, None]
        for c in range(nch):
            p = c % 2
            if puts[p] is not None:
                puts[p].wait()
            gets[p] = pltpu.async_copy(
                y_hbm.at[j_v.at[pl.ds(c * CH, CH)]], bufs[p], gsems[p])
            if c > 0:
                q = (c - 1) % 2
                gets[q].wait()
                puts[q] = pltpu.async_copy(
                    bufs[q], out_hbm.at[pl.ds(base + (c - 1) * CH, CH)],
                    ssems[q])
        last = (nch - 1) % 2
        gets[last].wait()
        puts[last] = pltpu.async_copy(
            bufs[last], out_hbm.at[pl.ds(base + (nch - 1) * CH, CH)],
            ssems[last])
        puts[0].wait()
        puts[1].wait()

    return k(y, jcat)


def _combine_kernel(a_ref, b_ref, tw0_ref, tw1_ref, o_ref):
    o_ref[...] = tw0_ref[...] * a_ref[...] + tw1_ref[...] * b_ref[...]


def _run_combine(yg, tw0, tw1):
    bs = 256
    return pl.pallas_call(
        _combine_kernel,
        grid=(S // bs,),
        in_specs=[
            pl.BlockSpec((bs, D), lambda i: (i, 0)),
            pl.BlockSpec((bs, D), lambda i: (i + S // bs, 0)),
            pl.BlockSpec((bs, 1), lambda i: (i, 0)),
            pl.BlockSpec((bs, 1), lambda i: (i, 0)),
        ],
        out_specs=pl.BlockSpec((bs, D), lambda i: (i, 0)),
        out_shape=jax.ShapeDtypeStruct((S, D), jnp.float32),
        compiler_params=pltpu.CompilerParams(
            dimension_semantics=("parallel",)),
    )(yg, yg, tw0, tw1)


def kernel(x, router_w, w1, v1, w2):
    x2 = x.reshape(S, D)
    weights, e0, e1, tw0, tw1 = _run_router(x2, router_w)
    block_expert, nb_used, jpos, tok = _routing_metadata(e0, e1)

    xg = _sc_dispatch(x2, tok, jpos.reshape(NW, NCH, CH))

    w1r = w1.reshape(E, FFN, D)
    v1r = v1.reshape(E, FFN, D)
    w2r = w2.reshape(E, FFN, D)
    y = _run_glu(block_expert, nb_used, xg, w1r, v1r, w2r)

    jcat = jpos.reshape(S, TOPK).T.reshape(TOPK * S)
    yg = _sc_collect(y, jcat)

    out = _run_combine(yg, tw0, tw1)
    return out.reshape(1, S, D), weights.reshape(1, S, E)


# speedup vs baseline: 1.2365x; 1.0450x over previous
import functools

import jax
import jax.numpy as jnp
from jax import lax
from jax.experimental import pallas as pl
from jax.experimental.pallas import tpu as pltpu
from jax.experimental.pallas import tpu_sc as plsc

E = 8
TOPK = 2
D = 1024
FFN = 1024
S = 2048

BK = 512
NA = TOPK * S
NB = NA // BK + E
NP = NB * BK
NW = 32
APW = NA // NW
CH = 32
NCH = APW // CH

_LANES = 128


def _router_kernel(x_ref, rw_ref, w_ref, e0_ref, e1_ref, tw0_ref, tw1_ref):
    x = x_ref[...]
    logits = jnp.dot(x, rw_ref[...], preferred_element_type=jnp.float32)
    lane = lax.broadcasted_iota(jnp.int32, logits.shape, 1)
    real = lane < E
    logits = jnp.where(real, logits, jnp.float32(-1e30))
    m = jnp.max(logits, axis=1, keepdims=True)
    p = jnp.where(real, jnp.exp(logits - m), 0.0)
    s = jnp.sum(p, axis=1, keepdims=True)
    w = p / s
    w_ref[...] = w[:, :E]
    w0 = jnp.max(w, axis=1, keepdims=True)
    e0 = jnp.min(jnp.where((w == w0) & real, lane, 2 * _LANES),
                 axis=1, keepdims=True)
    wm = jnp.where(lane == e0, jnp.float32(-1.0), w)
    w1v = jnp.max(wm, axis=1, keepdims=True)
    e1 = jnp.min(jnp.where((wm == w1v) & real, lane, 2 * _LANES),
                 axis=1, keepdims=True)
    tot = w0 + w1v
    e0_ref[...] = e0
    e1_ref[...] = e1
    tw0_ref[...] = w0 / tot
    tw1_ref[...] = w1v / tot


def _run_router(x2, router_w):
    rw_pad = jnp.zeros((D, _LANES), jnp.float32).at[:, :E].set(router_w)
    bs = 2048
    return pl.pallas_call(
        _router_kernel,
        grid=(S // bs,),
        in_specs=[
            pl.BlockSpec((bs, D), lambda i: (i, 0)),
            pl.BlockSpec((D, _LANES), lambda i: (0, 0)),
        ],
        out_specs=[
            pl.BlockSpec((bs, E), lambda i: (i, 0)),
            pl.BlockSpec((bs, 1), lambda i: (i, 0)),
            pl.BlockSpec((bs, 1), lambda i: (i, 0)),
            pl.BlockSpec((bs, 1), lambda i: (i, 0)),
            pl.BlockSpec((bs, 1), lambda i: (i, 0)),
        ],
        out_shape=[
            jax.ShapeDtypeStruct((S, E), jnp.float32),
            jax.ShapeDtypeStruct((S, 1), jnp.int32),
            jax.ShapeDtypeStruct((S, 1), jnp.int32),
            jax.ShapeDtypeStruct((S, 1), jnp.float32),
            jax.ShapeDtypeStruct((S, 1), jnp.float32),
        ],
        compiler_params=pltpu.CompilerParams(
            dimension_semantics=("parallel",)),
    )(x2, rw_pad)


def _routing_metadata(e0, e1):
    e_flat = jnp.concatenate([e0.reshape(S), e1.reshape(S)])
    onehot = (e_flat[:, None] == jnp.arange(E)[None, :]).astype(jnp.int32)
    csum = jnp.cumsum(onehot, axis=0)
    rank = jnp.sum(onehot * csum, axis=1) - 1
    counts = csum[-1]
    nb_e = ((counts + BK - 1) // BK).astype(jnp.int32)
    blk_start = (jnp.cumsum(nb_e) - nb_e).astype(jnp.int32)
    nb_used = jnp.sum(nb_e).astype(jnp.int32).reshape(1)
    block_expert = (jnp.sum(
        (jnp.arange(NB)[:, None] >= blk_start[None, :]).astype(jnp.int32),
        axis=1) - 1).astype(jnp.int32)
    start_a = jnp.sum(onehot * blk_start[None, :], axis=1)
    jpos = (start_a * BK + rank).astype(jnp.int32)
    tok = jnp.arange(NA, dtype=jnp.int32) % S
    return block_expert, nb_used, jpos, tok


_SC_MESH = dict(core_axis_name="c", subcore_axis_name="s")


def _sc_dispatch(x_f32, tok, jpos3):
    @functools.partial(
        pl.kernel,
        out_type=jax.ShapeDtypeStruct((NP, D), jnp.float32),
        mesh=plsc.VectorSubcoreMesh(**_SC_MESH),
        scratch_types=[
            pltpu.VMEM((APW,), jnp.int32),
            pltpu.VMEM((NCH, CH), jnp.int32),
            pltpu.VMEM((CH, D), jnp.float32),
            pltpu.VMEM((CH, D), jnp.float32),
            pltpu.SemaphoreType.DMA,
            pltpu.SemaphoreType.DMA,
            pltpu.SemaphoreType.DMA,
            pltpu.SemaphoreType.DMA,
        ],
    )
    def k(x_hbm, tok_hbm, jpos_hbm, out_hbm, tok_v, j_v, buf0, buf1,
          sg0, sg1, ss0, ss1):
        wid = lax.axis_index("s") * 2 + lax.axis_index("c")
        base = wid * APW
        pltpu.sync_copy(tok_hbm.at[pl.ds(base, APW)], tok_v)
        pltpu.sync_copy(jpos_hbm.at[wid], j_v)

        bufs = (buf0, buf1)
        gsems = (sg0, sg1)
        ssems = (ss0, ss1)
        gets = [None, None]
        puts = [None, None]
        for c in range(NCH):
            p = c % 2
            if puts[p] is not None:
                puts[p].wait()
            gets[p] = pltpu.async_copy(
                x_hbm.at[tok_v.at[pl.ds(c * CH, CH)]], bufs[p], gsems[p])
            if c > 0:
                q = (c - 1) % 2
                gets[q].wait()
                puts[q] = pltpu.async_copy(
                    bufs[q], out_hbm.at[j_v.at[c - 1]], ssems[q])
        last = (NCH - 1) % 2
        gets[last].wait()
        puts[last] = pltpu.async_copy(
            bufs[last], out_hbm.at[j_v.at[NCH - 1]], ssems[last])
        puts[0].wait()
        puts[1].wait()

    return k(x_f32, tok, jpos3)


def _glu_kernel(be_ref, nu_ref, xg_ref, w1_ref, v1_ref, w2_ref, y_ref):
    b = pl.program_id(0)

    @pl.when(b < nu_ref[0])
    def _():
        xb = xg_ref[...]
        h1 = lax.dot_general(xb, w1_ref[0], (((1,), (1,)), ((), ())),
                             preferred_element_type=jnp.float32,
                             precision=lax.Precision.DEFAULT)
        h2 = lax.dot_general(xb, v1_ref[0], (((1,), (1,)), ((), ())),
                             preferred_element_type=jnp.float32,
                             precision=lax.Precision.DEFAULT)
        g = h1 * lax.logistic(h1) * h2
        y = lax.dot_general(g, w2_ref[0], (((1,), (0,)), ((), ())),
                            preferred_element_type=jnp.float32,
                            precision=lax.Precision.DEFAULT)
        y_ref[...] = y


def _run_glu(block_expert, nb_used, xg, w1r, v1r, w2r):
    def _row_map(b, be, nu):
        return (jnp.minimum(b, nu[0] - 1), 0)

    def _w_map(b, be, nu):
        return (be[jnp.minimum(b, nu[0] - 1)], 0, 0)

    def _out_map(b, be, nu):
        return (jnp.where(b < nu[0], b, NB), 0)

    grid_spec = pltpu.PrefetchScalarGridSpec(
        num_scalar_prefetch=2,
        grid=(NB,),
        in_specs=[
            pl.BlockSpec((BK, D), _row_map),
            pl.BlockSpec((1, FFN, D), _w_map),
            pl.BlockSpec((1, FFN, D), _w_map),
            pl.BlockSpec((1, FFN, D), _w_map),
        ],
        out_specs=pl.BlockSpec((BK, D), _out_map),
    )
    return pl.pallas_call(
        _glu_kernel,
        grid_spec=grid_spec,
        out_shape=jax.ShapeDtypeStruct((NP + BK, D), jnp.float32),
    )(block_expert, nb_used, xg, w1r, v1r, w2r)


def _sc_collect(y, jcat):
    rpw = TOPK * S // NW
    nch = rpw // CH

    @functools.partial(
        pl.kernel,
        out_type=jax.ShapeDtypeStruct((TOPK * S, D), jnp.float32),
        mesh=plsc.VectorSubcoreMesh(**_SC_MESH),
        scratch_types=[
            pltpu.VMEM((rpw,), jnp.int32),
            pltpu.VMEM((CH, D), jnp.float32),
            pltpu.VMEM((CH, D), jnp.float32),
            pltpu.SemaphoreType.DMA,
            pltpu.SemaphoreType.DMA,
            pltpu.SemaphoreType.DMA,
            pltpu.SemaphoreType.DMA,
        ],
    )
    def k(y_hbm, j_hbm, out_hbm, j_v, buf0, buf1, sg0, sg1, ss0, ss1):
        wid = lax.axis_index("s") * 2 + lax.axis_index("c")
        base = wid * rpw
        pltpu.sync_copy(j_hbm.at[pl.ds(base, rpw)], j_v)

        bufs = (buf0, buf1)
        gsems = (sg0, sg1)
        ssems = (ss0, ss1)
        gets = [None, None]
        puts = [None, None]
        for c in range(nch):
            p = c % 2
            if puts[p] is not None:
                puts[p].wait()
            gets[p] = pltpu.async_copy(
                y_hbm.at[j_v.at[pl.ds(c * CH, CH)]], bufs[p], gsems[p])
            if c > 0:
                q = (c - 1) % 2
                gets[q].wait()
                puts[q] = pltpu.async_copy(
                    bufs[q], out_hbm.at[pl.ds(base + (c - 1) * CH, CH)],
                    ssems[q])
        last = (nch - 1) % 2
        gets[last].wait()
        puts[last] = pltpu.async_copy(
            bufs[last], out_hbm.at[pl.ds(base + (nch - 1) * CH, CH)],
            ssems[last])
        puts[0].wait()
        puts[1].wait()

    return k(y, jcat)


def _combine_kernel(a_ref, b_ref, tw0_ref, tw1_ref, o_ref):
    o_ref[...] = tw0_ref[...] * a_ref[...] + tw1_ref[...] * b_ref[...]


def _run_combine(yg, tw0, tw1):
    bs = 512
    return pl.pallas_call(
        _combine_kernel,
        grid=(S // bs,),
        in_specs=[
            pl.BlockSpec((bs, D), lambda i: (i, 0)),
            pl.BlockSpec((bs, D), lambda i: (i + S // bs, 0)),
            pl.BlockSpec((bs, 1), lambda i: (i, 0)),
            pl.BlockSpec((bs, 1), lambda i: (i, 0)),
        ],
        out_specs=pl.BlockSpec((bs, D), lambda i: (i, 0)),
        out_shape=jax.ShapeDtypeStruct((S, D), jnp.float32),
        compiler_params=pltpu.CompilerParams(
            dimension_semantics=("parallel",)),
    )(yg, yg, tw0, tw1)


def kernel(x, router_w, w1, v1, w2):
    x2 = x.reshape(S, D)
    weights, e0, e1, tw0, tw1 = _run_router(x2, router_w)
    block_expert, nb_used, jpos, tok = _routing_metadata(e0, e1)

    xg = _sc_dispatch(x2, tok, jpos.reshape(NW, NCH, CH))

    w1r = w1.reshape(E, FFN, D)
    v1r = v1.reshape(E, FFN, D)
    w2r = w2.reshape(E, FFN, D)
    y = _run_glu(block_expert, nb_used, xg, w1r, v1r, w2r)

    yg = _sc_collect(y, jpos)

    out = _run_combine(yg, tw0, tw1)
    return out.reshape(1, S, D), weights.reshape(1, S, E)


# speedup vs baseline: 1.2610x; 1.0198x over previous
import functools

import jax
import jax.numpy as jnp
from jax import lax
from jax.experimental import pallas as pl
from jax.experimental.pallas import tpu as pltpu
from jax.experimental.pallas import tpu_sc as plsc

E = 8
TOPK = 2
D = 1024
FFN = 1024
S = 2048

BK = 512
NA = TOPK * S
NB = NA // BK + E
NP = NB * BK
NW = 32
APW = NA // NW
CH = 32
NCH = APW // CH

_LANES = 128


def _router_kernel(x_ref, rw_ref, w_ref, e0_ref, e1_ref, tw0_ref, tw1_ref):
    x = x_ref[...]
    logits = jnp.dot(x, rw_ref[...], preferred_element_type=jnp.float32)
    lane = lax.broadcasted_iota(jnp.int32, logits.shape, 1)
    real = lane < E
    logits = jnp.where(real, logits, jnp.float32(-1e30))
    m = jnp.max(logits, axis=1, keepdims=True)
    p = jnp.where(real, jnp.exp(logits - m), 0.0)
    s = jnp.sum(p, axis=1, keepdims=True)
    w = p / s
    w_ref[...] = w[:, :E]
    w0 = jnp.max(w, axis=1, keepdims=True)
    e0 = jnp.min(jnp.where((w == w0) & real, lane, 2 * _LANES),
                 axis=1, keepdims=True)
    wm = jnp.where(lane == e0, jnp.float32(-1.0), w)
    w1v = jnp.max(wm, axis=1, keepdims=True)
    e1 = jnp.min(jnp.where((wm == w1v) & real, lane, 2 * _LANES),
                 axis=1, keepdims=True)
    tot = w0 + w1v
    e0_ref[...] = e0
    e1_ref[...] = e1
    tw0_ref[...] = w0 / tot
    tw1_ref[...] = w1v / tot


def _run_router(x2, router_w):
    rw_pad = jnp.zeros((D, _LANES), jnp.float32).at[:, :E].set(router_w)
    bs = 2048
    return pl.pallas_call(
        _router_kernel,
        grid=(S // bs,),
        in_specs=[
            pl.BlockSpec((bs, D), lambda i: (i, 0)),
            pl.BlockSpec((D, _LANES), lambda i: (0, 0)),
        ],
        out_specs=[
            pl.BlockSpec((bs, E), lambda i: (i, 0)),
            pl.BlockSpec((bs, 1), lambda i: (i, 0)),
            pl.BlockSpec((bs, 1), lambda i: (i, 0)),
            pl.BlockSpec((bs, 1), lambda i: (i, 0)),
            pl.BlockSpec((bs, 1), lambda i: (i, 0)),
        ],
        out_shape=[
            jax.ShapeDtypeStruct((S, E), jnp.float32),
            jax.ShapeDtypeStruct((S, 1), jnp.int32),
            jax.ShapeDtypeStruct((S, 1), jnp.int32),
            jax.ShapeDtypeStruct((S, 1), jnp.float32),
            jax.ShapeDtypeStruct((S, 1), jnp.float32),
        ],
        compiler_params=pltpu.CompilerParams(
            dimension_semantics=("parallel",)),
    )(x2, rw_pad)


def _routing_metadata(e0, e1):
    e_flat = jnp.concatenate([e0.reshape(S), e1.reshape(S)])
    onehot = (e_flat[:, None] == jnp.arange(E)[None, :]).astype(jnp.int32)
    csum = jnp.cumsum(onehot, axis=0)
    rank = jnp.sum(onehot * csum, axis=1) - 1
    counts = csum[-1]
    nb_e = ((counts + BK - 1) // BK).astype(jnp.int32)
    blk_start = (jnp.cumsum(nb_e) - nb_e).astype(jnp.int32)
    nb_used = jnp.sum(nb_e).astype(jnp.int32).reshape(1)
    block_expert = (jnp.sum(
        (jnp.arange(NB)[:, None] >= blk_start[None, :]).astype(jnp.int32),
        axis=1) - 1).astype(jnp.int32)
    start_a = jnp.sum(onehot * blk_start[None, :], axis=1)
    jpos = (start_a * BK + rank).astype(jnp.int32)
    tok = jnp.arange(NA, dtype=jnp.int32) % S

    barange = jnp.arange(NB)
    bvalid = barange < nb_used[0]
    prev = jnp.concatenate([jnp.full((1,), -1, jnp.int32), block_expert[:-1]])
    is_first = (bvalid & (block_expert != prev)).astype(jnp.int32)
    run_id = jnp.cumsum(is_first) - 1
    slot = (run_id % 2).astype(jnp.int32)
    ncp = jnp.sum((bvalid[None, :]
                   & (block_expert[None, :] <= block_expert[:, None]))
                  .astype(jnp.int32), axis=1)
    has_next = (ncp < nb_used[0]).astype(jnp.int32)
    ncp_c = jnp.minimum(ncp, NB - 1)
    next_e = jnp.sum((barange[None, :] == ncp_c[:, None]).astype(jnp.int32)
                     * block_expert[None, :], axis=1).astype(jnp.int32)
    sched = (nb_used, block_expert, is_first, slot, has_next, next_e)
    return sched, jpos, tok


_SC_MESH = dict(core_axis_name="c", subcore_axis_name="s")


def _sc_dispatch(x_f32, tok, jpos3):
    @functools.partial(
        pl.kernel,
        out_type=jax.ShapeDtypeStruct((NP, D), jnp.float32),
        mesh=plsc.VectorSubcoreMesh(**_SC_MESH),
        scratch_types=[
            pltpu.VMEM((APW,), jnp.int32),
            pltpu.VMEM((NCH, CH), jnp.int32),
            pltpu.VMEM((CH, D), jnp.float32),
            pltpu.VMEM((CH, D), jnp.float32),
            pltpu.SemaphoreType.DMA,
            pltpu.SemaphoreType.DMA,
            pltpu.SemaphoreType.DMA,
            pltpu.SemaphoreType.DMA,
        ],
    )
    def k(x_hbm, tok_hbm, jpos_hbm, out_hbm, tok_v, j_v, buf0, buf1,
          sg0, sg1, ss0, ss1):
        wid = lax.axis_index("s") * 2 + lax.axis_index("c")
        base = wid * APW
        pltpu.sync_copy(tok_hbm.at[pl.ds(base, APW)], tok_v)
        pltpu.sync_copy(jpos_hbm.at[wid], j_v)

        bufs = (buf0, buf1)
        gsems = (sg0, sg1)
        ssems = (ss0, ss1)
        gets = [None, None]
        puts = [None, None]
        for c in range(NCH):
            p = c % 2
            if puts[p] is not None:
                puts[p].wait()
            gets[p] = pltpu.async_copy(
                x_hbm.at[tok_v.at[pl.ds(c * CH, CH)]], bufs[p], gsems[p])
            if c > 0:
                q = (c - 1) % 2
                gets[q].wait()
                puts[q] = pltpu.async_copy(
                    bufs[q], out_hbm.at[j_v.at[c - 1]], ssems[q])
        last = (NCH - 1) % 2
        gets[last].wait()
        puts[last] = pltpu.async_copy(
            bufs[last], out_hbm.at[j_v.at[NCH - 1]], ssems[last])
        puts[0].wait()
        puts[1].wait()

    return k(x_f32, tok, jpos3)


def _glu_kernel(nu_ref, be_ref, isf_ref, sl_ref, hx_ref, ne_ref,
                xg_ref, w1_ref, v1_ref, w2_ref, y_ref,
                w1s, v1s, w2s, sem1, sem2, sem3):
    b = pl.program_id(0)

    def _issue(e, slot):
        pltpu.make_async_copy(w1_ref.at[pl.ds(e, 1)],
                              w1s.at[pl.ds(slot, 1)], sem1).start()
        pltpu.make_async_copy(v1_ref.at[pl.ds(e, 1)],
                              v1s.at[pl.ds(slot, 1)], sem2).start()
        pltpu.make_async_copy(w2_ref.at[pl.ds(e, 1)],
                              w2s.at[pl.ds(slot, 1)], sem3).start()

    def _wait():
        pltpu.make_async_copy(w1_ref.at[pl.ds(0, 1)],
                              w1s.at[pl.ds(0, 1)], sem1).wait()
        pltpu.make_async_copy(v1_ref.at[pl.ds(0, 1)],
                              v1s.at[pl.ds(0, 1)], sem2).wait()
        pltpu.make_async_copy(w2_ref.at[pl.ds(0, 1)],
                              w2s.at[pl.ds(0, 1)], sem3).wait()

    @pl.when(b == 0)
    def _():
        _issue(be_ref[0], 0)
        _wait()

    @pl.when((b < nu_ref[0]) & (isf_ref[b] == 1))
    def _():
        @pl.when(b > 0)
        def _():
            _wait()

        @pl.when(hx_ref[b] == 1)
        def _():
            _issue(ne_ref[b], 1 - sl_ref[b])

    @pl.when(b < nu_ref[0])
    def _():
        sl = sl_ref[b]
        xb = xg_ref[...]
        w1c = w1s[pl.ds(sl, 1)][0]
        v1c = v1s[pl.ds(sl, 1)][0]
        w2c = w2s[pl.ds(sl, 1)][0]
        h1 = lax.dot_general(xb, w1c, (((1,), (1,)), ((), ())),
                             preferred_element_type=jnp.float32,
                             precision=lax.Precision.DEFAULT)
        h2 = lax.dot_general(xb, v1c, (((1,), (1,)), ((), ())),
                             preferred_element_type=jnp.float32,
                             precision=lax.Precision.DEFAULT)
        g = h1 * lax.logistic(h1) * h2
        y = lax.dot_general(g, w2c, (((1,), (0,)), ((), ())),
                            preferred_element_type=jnp.float32,
                            precision=lax.Precision.DEFAULT)
        y_ref[...] = y


def _run_glu(sched, xg, w1r, v1r, w2r):
    def _row_map(b, nu, be, isf, sl, hx, ne):
        return (jnp.minimum(b, nu[0] - 1), 0)

    def _out_map(b, nu, be, isf, sl, hx, ne):
        return (jnp.where(b < nu[0], b, NB), 0)

    hbm = pl.BlockSpec(memory_space=pltpu.MemorySpace.HBM)
    grid_spec = pltpu.PrefetchScalarGridSpec(
        num_scalar_prefetch=6,
        grid=(NB,),
        in_specs=[
            pl.BlockSpec((BK, D), _row_map),
            hbm,
            hbm,
            hbm,
        ],
        out_specs=pl.BlockSpec((BK, D), _out_map),
        scratch_shapes=[
            pltpu.VMEM((2, FFN, D), jnp.float32),
            pltpu.VMEM((2, FFN, D), jnp.float32),
            pltpu.VMEM((2, FFN, D), jnp.float32),
            pltpu.SemaphoreType.DMA,
            pltpu.SemaphoreType.DMA,
            pltpu.SemaphoreType.DMA,
        ],
    )
    return pl.pallas_call(
        _glu_kernel,
        grid_spec=grid_spec,
        out_shape=jax.ShapeDtypeStruct((NP + BK, D), jnp.float32),
    )(*sched, xg, w1r, v1r, w2r)


def _sc_collect(y, jcat):
    rpw = TOPK * S // NW
    nch = rpw // CH

    @functools.partial(
        pl.kernel,
        out_type=jax.ShapeDtypeStruct((TOPK * S, D), jnp.float32),
        mesh=plsc.VectorSubcoreMesh(**_SC_MESH),
        scratch_types=[
            pltpu.VMEM((rpw,), jnp.int32),
            pltpu.VMEM((CH, D), jnp.float32),
            pltpu.VMEM((CH, D), jnp.float32),
            pltpu.SemaphoreType.DMA,
            pltpu.SemaphoreType.DMA,
            pltpu.SemaphoreType.DMA,
            pltpu.SemaphoreType.DMA,
        ],
    )
    def k(y_hbm, j_hbm, out_hbm, j_v, buf0, buf1, sg0, sg1, ss0, ss1):
        wid = lax.axis_index("s") * 2 + lax.axis_index("c")
        base = wid * rpw
        pltpu.sync_copy(j_hbm.at[pl.ds(base, rpw)], j_v)

        bufs = (buf0, buf1)
        gsems = (sg0, sg1)
        ssems = (ss0, ss1)
        gets = [None, None]
        puts = [None, None]
        for c in range(nch):
            p = c % 2
            if puts[p] is not None:
                puts[p].wait()
            gets[p] = pltpu.async_copy(
                y_hbm.at[j_v.at[pl.ds(c * CH, CH)]], bufs[p], gsems[p])
            if c > 0:
                q = (c - 1) % 2
                gets[q].wait()
                puts[q] = pltpu.async_copy(
                    bufs[q], out_hbm.at[pl.ds(base + (c - 1) * CH, CH)],
                    ssems[q])
        last = (nch - 1) % 2
        gets[last].wait()
        puts[last] = pltpu.async_copy(
            bufs[last], out_hbm.at[pl.ds(base + (nch - 1) * CH, CH)],
            ssems[last])
        puts[0].wait()
        puts[1].wait()

    return k(y, jcat)


def _combine_kernel(a_ref, b_ref, tw0_ref, tw1_ref, o_ref):
    o_ref[...] = tw0_ref[...] * a_ref[...] + tw1_ref[...] * b_ref[...]


def _run_combine(yg, tw0, tw1):
    bs = 512
    return pl.pallas_call(
        _combine_kernel,
        grid=(S // bs,),
        in_specs=[
            pl.BlockSpec((bs, D), lambda i: (i, 0)),
            pl.BlockSpec((bs, D), lambda i: (i + S // bs, 0)),
            pl.BlockSpec((bs, 1), lambda i: (i, 0)),
            pl.BlockSpec((bs, 1), lambda i: (i, 0)),
        ],
        out_specs=pl.BlockSpec((bs, D), lambda i: (i, 0)),
        out_shape=jax.ShapeDtypeStruct((S, D), jnp.float32),
        compiler_params=pltpu.CompilerParams(
            dimension_semantics=("parallel",)),
    )(yg, yg, tw0, tw1)


def kernel(x, router_w, w1, v1, w2):
    x2 = x.reshape(S, D)
    weights, e0, e1, tw0, tw1 = _run_router(x2, router_w)
    sched, jpos, tok = _routing_metadata(e0, e1)

    xg = _sc_dispatch(x2, tok, jpos.reshape(NW, NCH, CH))

    w1r = w1.reshape(E, FFN, D)
    v1r = v1.reshape(E, FFN, D)
    w2r = w2.reshape(E, FFN, D)
    y = _run_glu(sched, xg, w1r, v1r, w2r)

    yg = _sc_collect(y, jpos)

    out = _run_combine(yg, tw0, tw1)
    return out.reshape(1, S, D), weights.reshape(1, S, E)
